# baseline (speedup 1.0000x reference)
import jax
import jax.numpy as jnp
from jax.experimental import pallas as pl
from jax.experimental.pallas import tpu as pltpu

_BN = 2048


def _qtab_kernel(in_ref, mask_ref, table_ref, out_ref):
    i = pl.program_id(0)
    a = table_ref[...].astype(jnp.bfloat16)
    b = in_ref[...].astype(jnp.bfloat16)
    out_ref[:, pl.ds(i * _BN, _BN)] = (
        jnp.dot(a, b, preferred_element_type=jnp.float32)
        + mask_ref[:, pl.ds(i * _BN, _BN)]
    )


def kernel(inputs, mask, table):
    B, K = inputs.shape
    N = table.shape[1]
    out_t = pl.pallas_call(
        _qtab_kernel,
        grid=(B // _BN,),
        in_specs=[
            pl.BlockSpec((K, _BN), lambda i: (0, i)),
            pl.BlockSpec((N, B), lambda i: (0, 0)),
            pl.BlockSpec((N, K), lambda i: (0, 0)),
        ],
        out_specs=pl.BlockSpec((N, B), lambda i: (0, 0)),
        out_shape=jax.ShapeDtypeStruct((N, B), jnp.float32),
        compiler_params=pltpu.CompilerParams(
            dimension_semantics=("parallel",),
        ),
    )(inputs.T, mask.T, table.T)
    return out_t.T


# speedup vs baseline: 1.0096x; 1.0096x over previous
import jax
import jax.numpy as jnp
from jax.experimental import pallas as pl
from jax.experimental.pallas import tpu as pltpu

_BN = 2048


def _qtab_kernel(in_ref, mask_ref, table_ref, out_ref):
    i = pl.program_id(0)
    a = table_ref[...].astype(jnp.bfloat16)
    b = in_ref[...].astype(jnp.bfloat16)
    out_ref[:, pl.ds(i * _BN, _BN)] = (
        jnp.dot(a, b, preferred_element_type=jnp.float32)
        + mask_ref[:, pl.ds(i * _BN, _BN)]
    )


def kernel(inputs, mask, table):
    B, K = inputs.shape
    N = table.shape[1]
    out_t = pl.pallas_call(
        _qtab_kernel,
        grid=(B // _BN,),
        in_specs=[
            pl.BlockSpec((K, _BN), lambda i: (0, i)),
            pl.BlockSpec((N, B), lambda i: (0, 0)),
            pl.BlockSpec((N, K), lambda i: (0, 0)),
        ],
        out_specs=pl.BlockSpec((N, B), lambda i: (0, 0)),
        out_shape=jax.ShapeDtypeStruct((N, B), jnp.float32),
        compiler_params=pltpu.CompilerParams(
            dimension_semantics=("arbitrary",),
        ),
    )(inputs.T, mask.T, table.T)
    return out_t.T
